# baseline (speedup 1.0000x reference)
import functools

import jax
import jax.numpy as jnp
from jax import lax
from jax.experimental import pallas as pl
from jax.experimental.pallas import tpu as pltpu
from jax.experimental.pallas import tpu_sc as plsc

NC = 2
NS = 16
NW = NC * NS
CHUNK = 88
CHUNK_W = 128
WDEG = 16
FAST_C = 0
SHARE_AGG1 = 0.623
SHARE_AGG2 = 0.580


def _sc_mesh():
    return plsc.VectorSubcoreMesh(core_axis_name="c", subcore_axis_name="s")


def _make_deg_kernel(nacc, nch0, nch1, CHUNK=CHUNK):
    rpt = nacc // NS
    nch = max(nch0, nch1)

    @functools.partial(
        pl.kernel,
        out_type=jax.ShapeDtypeStruct((NC * nacc, WDEG), jnp.float32),
        mesh=_sc_mesh(),
        compiler_params=pltpu.CompilerParams(use_tc_tiling_on_sc=False),
        scratch_types=[
            pltpu.VMEM((nch, CHUNK), jnp.int32),
            pltpu.VMEM((CHUNK, WDEG), jnp.float32),
            pltpu.VMEM_SHARED((nacc, WDEG), jnp.float32),
        ],
    )
    def deg_kernel(dst_hbm, zeros_hbm, out_hbm, dst_v, ones_v, acc_sh):
        c = lax.axis_index("c")
        s = lax.axis_index("s")
        wid = s * NC + c
        pltpu.sync_copy(dst_hbm.at[wid], dst_v)

        def fill(i, carry):
            ones_v[i, :] = jnp.ones((WDEG,), jnp.float32)
            return carry

        lax.fori_loop(0, CHUNK, fill, 0)
        pltpu.sync_copy(zeros_hbm.at[pl.ds(s * rpt, rpt)],
                        acc_sh.at[pl.ds(s * rpt, rpt)])
        plsc.subcore_barrier()

        def body(j, carry):
            pltpu.sync_copy(ones_v, acc_sh.at[dst_v.at[j]], add=True)
            return carry

        lax.fori_loop(0, jnp.where(c == 0, nch0, nch1), body, 0)
        plsc.subcore_barrier()
        pltpu.sync_copy(acc_sh.at[pl.ds(s * rpt, rpt)],
                        out_hbm.at[pl.ds(c * nacc + s * rpt, rpt)])

    return deg_kernel


NBUF = 2


def _make_agg_kernel(nacc, nch0, nch1, width, CHUNK=CHUNK):
    rpt = nacc // NS
    nch = max(nch0, nch1)

    @functools.partial(
        pl.kernel,
        out_type=jax.ShapeDtypeStruct((NC * nacc, width), jnp.float32),
        mesh=_sc_mesh(),
        compiler_params=pltpu.CompilerParams(use_tc_tiling_on_sc=False),
        scratch_types=[
            pltpu.VMEM((nch, CHUNK), jnp.int32),
            pltpu.VMEM((nch, CHUNK), jnp.int32),
            pltpu.VMEM_SHARED((nacc, width), jnp.float32),
        ] + [pltpu.VMEM((CHUNK, width), jnp.float32) for _ in range(NBUF)]
          + [pltpu.SemaphoreType.DMA for _ in range(NBUF)],
    )
    def agg_kernel(hs_hbm, src_hbm, dst_hbm, out_hbm,
                   src_v, dst_v, acc_sh_ref, *bufs_and_sems):
        rows = list(bufs_and_sems[:NBUF])
        sems = list(bufs_and_sems[NBUF:])
        c = lax.axis_index("c")
        s = lax.axis_index("s")
        wid = s * NC + c
        pltpu.sync_copy(src_hbm.at[wid], src_v)
        pltpu.sync_copy(dst_hbm.at[wid], dst_v)

        zvec = jnp.zeros((16,), jnp.float32)

        def zfill(i, carry):
            for k in range(width // 16):
                rows[0][i, pl.ds(k * 16, 16)] = zvec
            return carry

        lax.fori_loop(0, CHUNK, zfill, 0)
        full, part = rpt // CHUNK, rpt % CHUNK
        for t in range(full):
            pltpu.sync_copy(rows[0],
                            acc_sh_ref.at[pl.ds(s * rpt + t * CHUNK, CHUNK)])
        if part:
            pltpu.sync_copy(rows[0].at[pl.ds(0, part)],
                            acc_sh_ref.at[pl.ds(s * rpt + full * CHUNK, part)])
        plsc.subcore_barrier()

        nch_dyn = jnp.where(c == 0, nch0, nch1)
        for b in range(min(NBUF, min(nch0, nch1))):
            pltpu.async_copy(hs_hbm.at[src_v.at[b]], rows[b], sems[b])

        def step(j, b):
            pltpu.make_async_copy(hs_hbm.at[src_v.at[0]], rows[b], sems[b]).wait()
            pltpu.sync_copy(rows[b], acc_sh_ref.at[dst_v.at[j]], add=True)

            @pl.when(j + NBUF < nch_dyn)
            def _():
                pltpu.async_copy(hs_hbm.at[src_v.at[j + NBUF]], rows[b], sems[b])

        def body(g, carry):
            for b in range(NBUF):
                step(g * NBUF + b, b)
            return carry

        lax.fori_loop(0, nch_dyn // NBUF, body, 0)

        plsc.subcore_barrier()
        for t in range(full):
            pltpu.sync_copy(acc_sh_ref.at[pl.ds(s * rpt + t * CHUNK, CHUNK)],
                            rows[0])
            pltpu.sync_copy(rows[0],
                            out_hbm.at[pl.ds(c * nacc + s * rpt + t * CHUNK,
                                             CHUNK)])
        if part:
            pltpu.sync_copy(acc_sh_ref.at[pl.ds(s * rpt + full * CHUNK, part)],
                            rows[0].at[pl.ds(0, part)])
            pltpu.sync_copy(rows[0].at[pl.ds(0, part)],
                            out_hbm.at[pl.ds(c * nacc + s * rpt + full * CHUNK,
                                             part)])

    return agg_kernel


def _make_layer1_body(n, nacc):
    def body(x_ref, w_ref, deg_ref, hs_ref, dinv_ref):
        deg = (deg_ref[pl.ds(0, n), pl.ds(0, 1)]
               + deg_ref[pl.ds(nacc, n), pl.ds(0, 1)] + 1.0)
        dinv = lax.rsqrt(deg)
        h = jnp.dot(x_ref[...], w_ref[...], preferred_element_type=jnp.float32)
        hs_ref[...] = h * dinv
        dinv_ref[...] = dinv
    return body


def _make_layer2_body(n, nacc):
    def body(agg_ref, hs1_ref, dinv_ref, b1_ref, w2_ref, hs2_ref):
        dinv = dinv_ref[...]
        acc = agg_ref[pl.ds(0, n), :] + agg_ref[pl.ds(nacc, n), :]
        x1 = jnp.maximum(dinv * (acc + hs1_ref[...]) + b1_ref[...], 0.0)
        hs2_ref[...] = jnp.dot(
            x1, w2_ref[...], preferred_element_type=jnp.float32) * dinv
    return body


def _make_out_body(n, nacc):
    def body(agg_ref, hs2_ref, dinv_ref, b2_ref, out_ref):
        acc = agg_ref[pl.ds(0, n), :] + agg_ref[pl.ds(nacc, n), :]
        z = dinv_ref[...] * (acc + hs2_ref[...]) + b2_ref[...]
        m = jnp.max(z, axis=1, keepdims=True)
        zs = z - m
        out_ref[...] = zs - jnp.log(jnp.sum(jnp.exp(zs), axis=1, keepdims=True))
    return body


def kernel(features, edge_index, W1, b1, W2, b2):
    n, f_in = features.shape
    hid = W1.shape[1]
    cdim = W2.shape[1]
    e = edge_index.shape[1]

    nacc = (n // (NS * 8) + 1) * (NS * 8)

    def partition(share, chunk=CHUNK):
        ntot = 2 * ((e + 2 * NS * chunk - 1) // (2 * NS * chunk))
        nchf = int(round(ntot * share / 2)) * 2
        nchs = ntot - nchf
        ef = NS * nchf * chunk
        pad = NS * ntot * chunk - e

        def part(idx, fill):
            idx_p = jnp.concatenate([idx, jnp.full((pad,), fill, jnp.int32)])
            f = idx_p[:ef].reshape(NS, nchf, chunk)
            sl = idx_p[ef:].reshape(NS, nchs, chunk)
            sl = jnp.pad(sl, ((0, 0), (0, nchf - nchs), (0, 0)),
                         constant_values=fill)
            pair = (f, sl) if FAST_C == 0 else (sl, f)
            return jnp.stack(pair, axis=1).reshape(NW, nchf, chunk)

        nch0, nch1 = (nchf, nchs) if FAST_C == 0 else (nchs, nchf)
        return nch0, nch1, part(edge_index[0], 0), part(edge_index[1], n)

    nch0, nch1, src_p, dst_p = partition(SHARE_AGG1)
    nch0b, nch1b, src_pb, dst_pb = partition(SHARE_AGG2, CHUNK_W)

    zeros_deg = jnp.zeros((nacc, WDEG), jnp.float32)

    deg_out = _make_deg_kernel(nacc, nch0, nch1)(dst_p, zeros_deg)

    hs1, dinv = pl.pallas_call(
        _make_layer1_body(n, nacc),
        out_shape=[
            jax.ShapeDtypeStruct((n, hid), jnp.float32),
            jax.ShapeDtypeStruct((n, 1), jnp.float32),
        ],
    )(features, W1, deg_out)

    agg1 = _make_agg_kernel(nacc, nch0, nch1, hid)(hs1, src_p, dst_p)

    hs2 = pl.pallas_call(
        _make_layer2_body(n, nacc),
        out_shape=jax.ShapeDtypeStruct((n, cdim), jnp.float32),
    )(agg1, hs1, dinv, b1.reshape(1, hid), W2)

    agg2 = _make_agg_kernel(nacc, nch0b, nch1b, cdim, CHUNK_W)(hs2, src_pb, dst_pb)

    out = pl.pallas_call(
        _make_out_body(n, nacc),
        out_shape=jax.ShapeDtypeStruct((n, cdim), jnp.float32),
    )(agg2, hs2, dinv, b2.reshape(1, cdim))

    return out


# speedup vs baseline: 1.1242x; 1.1242x over previous
import functools

import jax
import jax.numpy as jnp
from jax import lax
from jax.experimental import pallas as pl
from jax.experimental.pallas import tpu as pltpu
from jax.experimental.pallas import tpu_sc as plsc

NC = 2
NS = 16
NW = NC * NS
CHUNK = 88
CHUNK_W = 88
WDEG = 16
FAST_C = 0
SHARE_AGG1 = 0.623
SHARE_AGG2 = 0.580


def _sc_mesh():
    return plsc.VectorSubcoreMesh(core_axis_name="c", subcore_axis_name="s")


def _make_deg_kernel(nacc, nch0, nch1, CHUNK=CHUNK):
    rpt = nacc // NS
    nch = max(nch0, nch1)

    @functools.partial(
        pl.kernel,
        out_type=jax.ShapeDtypeStruct((NC * nacc, WDEG), jnp.float32),
        mesh=_sc_mesh(),
        compiler_params=pltpu.CompilerParams(use_tc_tiling_on_sc=False),
        scratch_types=[
            pltpu.VMEM((nch, CHUNK), jnp.int32),
            pltpu.VMEM((CHUNK, WDEG), jnp.float32),
            pltpu.VMEM_SHARED((nacc, WDEG), jnp.float32),
        ],
    )
    def deg_kernel(dst_hbm, zeros_hbm, out_hbm, dst_v, ones_v, acc_sh):
        c = lax.axis_index("c")
        s = lax.axis_index("s")
        wid = s * NC + c
        pltpu.sync_copy(dst_hbm.at[wid], dst_v)

        def fill(i, carry):
            ones_v[i, :] = jnp.ones((WDEG,), jnp.float32)
            return carry

        lax.fori_loop(0, CHUNK, fill, 0)
        pltpu.sync_copy(zeros_hbm.at[pl.ds(s * rpt, rpt)],
                        acc_sh.at[pl.ds(s * rpt, rpt)])
        plsc.subcore_barrier()

        def body(j, carry):
            pltpu.sync_copy(ones_v, acc_sh.at[dst_v.at[j]], add=True)
            return carry

        lax.fori_loop(0, jnp.where(c == 0, nch0, nch1), body, 0)
        plsc.subcore_barrier()
        pltpu.sync_copy(acc_sh.at[pl.ds(s * rpt, rpt)],
                        out_hbm.at[pl.ds(c * nacc + s * rpt, rpt)])

    return deg_kernel


NBUF = 2


def _make_agg_kernel(nacc, nch0, nch1, width, CHUNK=CHUNK):
    rpt = nacc // NS
    nch = max(nch0, nch1)

    @functools.partial(
        pl.kernel,
        out_type=jax.ShapeDtypeStruct((NC * nacc, width), jnp.float32),
        mesh=_sc_mesh(),
        compiler_params=pltpu.CompilerParams(use_tc_tiling_on_sc=False),
        scratch_types=[
            pltpu.VMEM((nch, CHUNK), jnp.int32),
            pltpu.VMEM((nch, CHUNK), jnp.int32),
            pltpu.VMEM_SHARED((nacc, width), jnp.float32),
        ] + [pltpu.VMEM((CHUNK, width), jnp.float32) for _ in range(NBUF)]
          + [pltpu.SemaphoreType.DMA for _ in range(NBUF)],
    )
    def agg_kernel(hs_hbm, src_hbm, dst_hbm, out_hbm,
                   src_v, dst_v, acc_sh_ref, *bufs_and_sems):
        rows = list(bufs_and_sems[:NBUF])
        sems = list(bufs_and_sems[NBUF:])
        c = lax.axis_index("c")
        s = lax.axis_index("s")
        wid = s * NC + c
        pltpu.sync_copy(src_hbm.at[wid], src_v)
        pltpu.sync_copy(dst_hbm.at[wid], dst_v)

        zvec = jnp.zeros((16,), jnp.float32)

        def zfill(i, carry):
            for k in range(width // 16):
                rows[0][i, pl.ds(k * 16, 16)] = zvec
            return carry

        lax.fori_loop(0, CHUNK, zfill, 0)
        full, part = rpt // CHUNK, rpt % CHUNK
        for t in range(full):
            pltpu.sync_copy(rows[0],
                            acc_sh_ref.at[pl.ds(s * rpt + t * CHUNK, CHUNK)])
        if part:
            pltpu.sync_copy(rows[0].at[pl.ds(0, part)],
                            acc_sh_ref.at[pl.ds(s * rpt + full * CHUNK, part)])
        plsc.subcore_barrier()

        nch_dyn = jnp.where(c == 0, nch0, nch1)
        for b in range(min(NBUF, min(nch0, nch1))):
            pltpu.async_copy(hs_hbm.at[src_v.at[b]], rows[b], sems[b])

        def step(j, b):
            pltpu.make_async_copy(hs_hbm.at[src_v.at[0]], rows[b], sems[b]).wait()
            pltpu.sync_copy(rows[b], acc_sh_ref.at[dst_v.at[j]], add=True)

            @pl.when(j + NBUF < nch_dyn)
            def _():
                pltpu.async_copy(hs_hbm.at[src_v.at[j + NBUF]], rows[b], sems[b])

        def body(g, carry):
            for b in range(NBUF):
                step(g * NBUF + b, b)
            return carry

        lax.fori_loop(0, nch_dyn // NBUF, body, 0)

        plsc.subcore_barrier()
        for t in range(full):
            pltpu.sync_copy(acc_sh_ref.at[pl.ds(s * rpt + t * CHUNK, CHUNK)],
                            rows[0])
            pltpu.sync_copy(rows[0],
                            out_hbm.at[pl.ds(c * nacc + s * rpt + t * CHUNK,
                                             CHUNK)])
        if part:
            pltpu.sync_copy(acc_sh_ref.at[pl.ds(s * rpt + full * CHUNK, part)],
                            rows[0].at[pl.ds(0, part)])
            pltpu.sync_copy(rows[0].at[pl.ds(0, part)],
                            out_hbm.at[pl.ds(c * nacc + s * rpt + full * CHUNK,
                                             part)])

    return agg_kernel


def _make_layer1_body(n, nacc):
    def body(x_ref, w_ref, deg_ref, hs_ref, dinv_ref):
        deg = (deg_ref[pl.ds(0, n), pl.ds(0, 1)]
               + deg_ref[pl.ds(nacc, n), pl.ds(0, 1)] + 1.0)
        dinv = lax.rsqrt(deg)
        h = jnp.dot(x_ref[...], w_ref[...], preferred_element_type=jnp.float32)
        hs_ref[...] = h * dinv
        dinv_ref[...] = dinv
    return body


def _make_layer2_body(n, nacc):
    def body(agg_ref, hs1_ref, dinv_ref, b1_ref, w2_ref, hs2_ref):
        dinv = dinv_ref[...]
        acc = agg_ref[pl.ds(0, n), :] + agg_ref[pl.ds(nacc, n), :]
        x1 = jnp.maximum(dinv * (acc + hs1_ref[...]) + b1_ref[...], 0.0)
        hs2_ref[...] = jnp.dot(
            x1, w2_ref[...], preferred_element_type=jnp.float32) * dinv
    return body


def _make_out_body(n, nacc):
    def body(agg_ref, hs2_ref, dinv_ref, b2_ref, out_ref):
        acc = agg_ref[pl.ds(0, n), :] + agg_ref[pl.ds(nacc, n), :]
        z = dinv_ref[...] * (acc + hs2_ref[...]) + b2_ref[...]
        m = jnp.max(z, axis=1, keepdims=True)
        zs = z - m
        out_ref[...] = zs - jnp.log(jnp.sum(jnp.exp(zs), axis=1, keepdims=True))
    return body


def kernel(features, edge_index, W1, b1, W2, b2):
    n, f_in = features.shape
    hid = W1.shape[1]
    cdim = W2.shape[1]
    e = edge_index.shape[1]

    nacc = (n // (NS * 8) + 1) * (NS * 8)

    def partition(share, chunk=CHUNK):
        ntot = 2 * ((e + 2 * NS * chunk - 1) // (2 * NS * chunk))
        nchf = int(round(ntot * share / 2)) * 2
        nchs = ntot - nchf
        ef = NS * nchf * chunk
        pad = NS * ntot * chunk - e

        def part(idx, fill):
            idx_p = jnp.concatenate([idx, jnp.full((pad,), fill, jnp.int32)])
            f = idx_p[:ef].reshape(NS, nchf, chunk)
            sl = idx_p[ef:].reshape(NS, nchs, chunk)
            sl = jnp.pad(sl, ((0, 0), (0, nchf - nchs), (0, 0)),
                         constant_values=fill)
            pair = (f, sl) if FAST_C == 0 else (sl, f)
            return jnp.stack(pair, axis=1).reshape(NW, nchf, chunk)

        nch0, nch1 = (nchf, nchs) if FAST_C == 0 else (nchs, nchf)
        return nch0, nch1, part(edge_index[0], 0), part(edge_index[1], n)

    nch0, nch1, src_p, dst_p = partition(SHARE_AGG1)
    nch0b, nch1b, src_pb, dst_pb = partition(SHARE_AGG2, CHUNK_W)

    zeros_deg = jnp.zeros((nacc, WDEG), jnp.float32)

    deg_out = _make_deg_kernel(nacc, nch0, nch1)(dst_p, zeros_deg)

    hs1, dinv = pl.pallas_call(
        _make_layer1_body(n, nacc),
        out_shape=[
            jax.ShapeDtypeStruct((n, hid), jnp.float32),
            jax.ShapeDtypeStruct((n, 1), jnp.float32),
        ],
    )(features, W1, deg_out)

    agg1 = _make_agg_kernel(nacc, nch0, nch1, hid)(hs1, src_p, dst_p)

    hs2 = pl.pallas_call(
        _make_layer2_body(n, nacc),
        out_shape=jax.ShapeDtypeStruct((n, cdim), jnp.float32),
    )(agg1, hs1, dinv, b1.reshape(1, hid), W2)

    agg2 = _make_agg_kernel(nacc, nch0b, nch1b, cdim, CHUNK_W)(hs2, src_pb, dst_pb)

    out = pl.pallas_call(
        _make_out_body(n, nacc),
        out_shape=jax.ShapeDtypeStruct((n, cdim), jnp.float32),
    )(agg2, hs2, dinv, b2.reshape(1, cdim))

    return out
